# baseline (speedup 1.0000x reference)
import functools

import jax
import jax.numpy as jnp
from jax import lax
from jax.experimental import pallas as pl
from jax.experimental.pallas import tpu as pltpu
from jax.experimental.pallas import tpu_sc as plsc

_F, _N, _B, _C = 2, 2, 16384, 1000
_M = 0.3
_T = 0.1
_LAMBDA_P = 0.1
_RHO = 0.01

_NS = _F * _N
_NEG = -1e10
_INV = 1.0 / (_NS * _B)

_CM = 200
_NJ = _C // _CM
_NTC = _NS - 1

_NC, _NSUB, _L = 2, 16, 16
_NW = _NC * _NSUB
_BPW = _B // _NW
_NG = _BPW // _L
_SCM = 40
_NCH = _C // _SCM
_UNROLL = 10


def _tc_body(y_ref, lab_ref, out_ref, fy_scr, mx_scr):
    s = pl.program_id(0)
    j = pl.program_id(1)

    @pl.when((s == 0) & (j == 0))
    def _init():
        out_ref[0, 0] = 0.0

    @pl.when(j == 0)
    def _reset():
        fy_scr[...] = jnp.zeros((1, _B), jnp.float32)
        mx_scr[...] = jnp.full((1, _B), _NEG, jnp.float32)

    yb = y_ref[0, 0]
    lab = lab_ref[...]
    crow = jax.lax.broadcasted_iota(jnp.int32, (_CM, _B), 0) + j * _CM
    mask = crow == lab
    fy_scr[...] += jnp.sum(jnp.where(mask, yb, 0.0), axis=0, keepdims=True)
    blk_mx = jnp.max(jnp.where(mask, _NEG, yb), axis=0, keepdims=True)
    mx_scr[...] = jnp.maximum(mx_scr[...], blk_mx)

    @pl.when(j == _NJ - 1)
    def _slice_done():
        l = jnp.maximum(_M + _T - fy_scr[...], 0.0) + jnp.maximum(
            _M + mx_scr[...], 0.0
        )
        out_ref[0, 0] += jnp.sum(l) * _INV


def _tc_partial(yt, lab2):
    return pl.pallas_call(
        _tc_body,
        grid=(_NTC, _NJ),
        in_specs=[
            pl.BlockSpec(
                (1, 1, _CM, _B),
                lambda s, j: ((s + 1) // _N, (s + 1) % _N, j, 0),
            ),
            pl.BlockSpec((1, _B), lambda s, j: (0, 0)),
        ],
        out_specs=pl.BlockSpec(memory_space=pltpu.SMEM),
        out_shape=jax.ShapeDtypeStruct((1, 1), jnp.float32),
        scratch_shapes=[
            pltpu.VMEM((1, _B), jnp.float32),
            pltpu.VMEM((1, _B), jnp.float32),
        ],
        compiler_params=pltpu.CompilerParams(
            dimension_semantics=("arbitrary", "arbitrary"),
        ),
    )(yt, lab2)


def _sc_chunk(buf, lab_v, facc_v, macc_v, ch):
    c0 = ch * _SCM
    for g in range(_NG):
        lanes = lax.broadcasted_iota(jnp.int32, (_L,), 0) + g * _L
        labs = lab_v[pl.ds(g * _L, _L)]
        rel = labs - c0
        inb = (rel >= 0) & (rel < _SCM)
        relc = jnp.clip(rel, 0, _SCM - 1)
        vals = plsc.load_gather(buf, [relc, lanes])
        facc_v[pl.ds(g * _L, _L)] = facc_v[pl.ds(g * _L, _L)] + jnp.where(
            inb, vals, 0.0
        )
        plsc.store_scatter(
            buf, [relc, lanes], jnp.full((_L,), _NEG, jnp.float32), mask=inb
        )

        def col_body(k, m):
            for u in range(_UNROLL):
                m = jnp.maximum(m, buf[k * _UNROLL + u, pl.ds(g * _L, _L)])
            return m

        m = lax.fori_loop(0, _SCM // _UNROLL, col_body, macc_v[pl.ds(g * _L, _L)])
        macc_v[pl.ds(g * _L, _L)] = m


def _sc_partials(yt, label):
    mesh = plsc.VectorSubcoreMesh(core_axis_name="c", subcore_axis_name="s")

    @functools.partial(
        pl.kernel,
        mesh=mesh,
        out_type=jax.ShapeDtypeStruct((_NW, _L), jnp.float32),
        scratch_types=[
            pltpu.VMEM((_BPW,), jnp.int32),
            pltpu.VMEM((_SCM, _BPW), jnp.float32),
            pltpu.VMEM((_SCM, _BPW), jnp.float32),
            pltpu.VMEM((_BPW,), jnp.float32),
            pltpu.VMEM((_BPW,), jnp.float32),
            pltpu.VMEM((_L,), jnp.float32),
            pltpu.SemaphoreType.DMA,
            pltpu.SemaphoreType.DMA,
        ],
        compiler_params=pltpu.CompilerParams(needs_layout_passes=False),
    )
    def sc_k(y_hbm, lab_hbm, out_hbm, lab_v, buf0, buf1, facc_v, macc_v,
             part_v, sem0, sem1):
        wid = lax.axis_index("s") * _NC + lax.axis_index("c")
        b0 = wid * _BPW

        pltpu.sync_copy(lab_hbm.at[pl.ds(b0, _BPW)], lab_v)
        for g in range(_NG):
            facc_v[pl.ds(g * _L, _L)] = jnp.zeros((_L,), jnp.float32)
            macc_v[pl.ds(g * _L, _L)] = jnp.full((_L,), _NEG, jnp.float32)

        def src(ch):
            return y_hbm.at[0, 0, pl.ds(ch * _SCM, _SCM), pl.ds(b0, _BPW)]

        pltpu.make_async_copy(src(0), buf0, sem0).start()

        def step(j, carry):
            pltpu.make_async_copy(src(2 * j + 1), buf1, sem1).start()
            pltpu.make_async_copy(src(2 * j), buf0, sem0).wait()
            _sc_chunk(buf0, lab_v, facc_v, macc_v, 2 * j)
            pltpu.make_async_copy(src(2 * j + 2), buf0, sem0).start()
            pltpu.make_async_copy(src(2 * j + 1), buf1, sem1).wait()
            _sc_chunk(buf1, lab_v, facc_v, macc_v, 2 * j + 1)
            return carry

        lax.fori_loop(0, _NCH // 2, step, jnp.int32(0))
        pltpu.make_async_copy(src(_NCH - 1), buf0, sem0).wait()
        _sc_chunk(buf0, lab_v, facc_v, macc_v, _NCH - 1)

        hacc = jnp.zeros((_L,), jnp.float32)
        for g in range(_NG):
            f16 = facc_v[pl.ds(g * _L, _L)]
            m16 = macc_v[pl.ds(g * _L, _L)]
            hacc = hacc + jnp.maximum(_M + _T - f16, 0.0)
            hacc = hacc + jnp.maximum(_M + m16, 0.0)
        part_v[...] = hacc
        pltpu.sync_copy(part_v, out_hbm.at[wid])

    return sc_k(yt, label)


def _combine_body(p_ref, tc_ref, pc_ref, out_ref):
    pc = pc_ref[0, 0]
    total = tc_ref[0, 0] + jnp.sum(p_ref[...]) * _INV
    out_ref[0, 0] = total + _LAMBDA_P * pc + (_RHO / 2.0) * pc * pc


def kernel(y, label, power_ratio, power_consumption):
    del power_ratio
    yt = jnp.transpose(y, (0, 1, 3, 2))
    lab2 = label[None, :]
    pc = power_consumption.reshape(1, 1)

    sc_parts = _sc_partials(yt, label)
    tc_part = _tc_partial(yt, lab2)

    out = pl.pallas_call(
        _combine_body,
        in_specs=[
            pl.BlockSpec((_NW, _L), lambda: (0, 0)),
            pl.BlockSpec(memory_space=pltpu.SMEM),
            pl.BlockSpec(memory_space=pltpu.SMEM),
        ],
        out_specs=pl.BlockSpec(memory_space=pltpu.SMEM),
        out_shape=jax.ShapeDtypeStruct((1, 1), jnp.float32),
    )(sc_parts, tc_part, pc)
    return out.reshape(1)


# speedup vs baseline: 1.3663x; 1.3663x over previous
import jax
import jax.numpy as jnp
from jax.experimental import pallas as pl
from jax.experimental.pallas import tpu as pltpu

_F, _N, _B, _C = 2, 2, 16384, 1000
_M = 0.3
_T = 0.1
_LAMBDA_P = 0.1
_RHO = 0.01

_CM = 200
_NJ = _C // _CM
_NS = _F * _N
_NEG = -1e10


def _loss_body(ya_ref, yb_ref, lab_ref, pc_ref, out_ref, fy_scr, mx_scr):
    s = pl.program_id(0)
    j = pl.program_id(1)

    @pl.when((s == 0) & (j == 0))
    def _init():
        out_ref[0, 0] = 0.0

    @pl.when(j == 0)
    def _reset():
        fy_scr[...] = jnp.zeros((1, _B), jnp.float32)
        mx_scr[...] = jnp.full((1, _B), _NEG, jnp.float32)

    lab = lab_ref[...]
    crow = jax.lax.broadcasted_iota(jnp.int32, (_CM, _B // 2), 0) + j * _CM
    for k, y_ref in enumerate((ya_ref, yb_ref)):
        yb = y_ref[0, 0]
        lo, hi = k * (_B // 2), (k + 1) * (_B // 2)
        mask = crow == lab[:, lo:hi]
        fy_scr[:, lo:hi] += jnp.sum(jnp.where(mask, yb, 0.0), axis=0, keepdims=True)
        blk_mx = jnp.max(jnp.where(mask, _NEG, yb), axis=0, keepdims=True)
        mx_scr[:, lo:hi] = jnp.maximum(mx_scr[:, lo:hi], blk_mx)

    @pl.when(j == _NJ - 1)
    def _slice_done():
        l = jnp.maximum(_M + _T - fy_scr[...], 0.0) + jnp.maximum(
            _M + mx_scr[...], 0.0
        )
        out_ref[0, 0] += jnp.sum(l) * (1.0 / (_NS * _B))

    @pl.when((s == _NS - 1) & (j == _NJ - 1))
    def _fini():
        pc = pc_ref[0, 0]
        out_ref[0, 0] += _LAMBDA_P * pc + (_RHO / 2.0) * pc * pc


def kernel(y, label, power_ratio, power_consumption):
    del power_ratio
    yt = jnp.transpose(y, (0, 1, 3, 2))
    lab2 = label[None, :]
    pc = power_consumption.reshape(1, 1)

    out = pl.pallas_call(
        _loss_body,
        grid=(_NS, _NJ),
        in_specs=[
            pl.BlockSpec(
                (1, 1, _CM, _B // 2), lambda s, j: (s // _N, s % _N, j, 0)
            ),
            pl.BlockSpec(
                (1, 1, _CM, _B // 2), lambda s, j: (s // _N, s % _N, j, 1)
            ),
            pl.BlockSpec((1, _B), lambda s, j: (0, 0)),
            pl.BlockSpec(memory_space=pltpu.SMEM),
        ],
        out_specs=pl.BlockSpec(memory_space=pltpu.SMEM),
        out_shape=jax.ShapeDtypeStruct((1, 1), jnp.float32),
        scratch_shapes=[
            pltpu.VMEM((1, _B), jnp.float32),
            pltpu.VMEM((1, _B), jnp.float32),
        ],
        compiler_params=pltpu.CompilerParams(
            dimension_semantics=("arbitrary", "arbitrary"),
        ),
    )(yt, yt, lab2, pc)
    return out.reshape(1)
